# Initial kernel scaffold; baseline (speedup 1.0000x reference)
import jax
import jax.numpy as jnp
from jax.experimental import pallas as pl


def kernel(node_features, edge_features, edge_index, last_latent, W_node, W_edge, W_msg, W_upd, W_dec, W_term):
    raise NotImplementedError("write your pallas kernel here")


# speedup vs baseline: 4.6026x; 4.6026x over previous
import functools

import jax
import jax.numpy as jnp
from jax import lax
from jax.experimental import pallas as pl
from jax.experimental.pallas import tpu as pltpu
from jax.experimental.pallas import tpu_sc as plsc

_L = 32
_CH = 256
_NSUB = 16
_NCORE = 2


def _encode_body(x_ref, wn_ref, ws_ref, wd_ref, ne_ref, ps_ref, pd_ref):
    x = x_ref[...]
    y = jnp.dot(x, wn_ref[...], preferred_element_type=jnp.float32, precision=lax.Precision.HIGHEST)
    ne = jnp.where(y >= 0, y, 0.01 * y)
    ne_ref[...] = ne
    ps_ref[...] = jnp.dot(ne, ws_ref[...], preferred_element_type=jnp.float32, precision=lax.Precision.HIGHEST)
    pd_ref[...] = jnp.dot(ne, wd_ref[...], preferred_element_type=jnp.float32, precision=lax.Precision.HIGHEST)


def _encode(x, wn, ws, wd, bn):
    n, nfl = x.shape
    grid = n // bn
    out = jax.ShapeDtypeStruct((n, _L), jnp.float32)
    return pl.pallas_call(
        _encode_body,
        grid=(grid,),
        in_specs=[
            pl.BlockSpec((bn, nfl), lambda i: (i, 0)),
            pl.BlockSpec((nfl, _L), lambda i: (0, 0)),
            pl.BlockSpec((_L, _L), lambda i: (0, 0)),
            pl.BlockSpec((_L, _L), lambda i: (0, 0)),
        ],
        out_specs=[
            pl.BlockSpec((bn, _L), lambda i: (i, 0)),
            pl.BlockSpec((bn, _L), lambda i: (i, 0)),
            pl.BlockSpec((bn, _L), lambda i: (i, 0)),
        ],
        out_shape=[out, out, out],
    )(x, wn, ws, wd)


def _decode_body(n_total, ne_ref, ag_ref, wu1_ref, wu2_ref, wd1_ref, wd2_ref,
                 wt_ref, out_ref, lat_ref, term_ref, acc_ref):
    i = pl.program_id(0)
    ng = pl.num_programs(0)
    ne = ne_ref[...]
    ag = ag_ref[...]
    lat = jnp.dot(ne, wu1_ref[...], preferred_element_type=jnp.float32, precision=lax.Precision.HIGHEST)
    lat += jnp.dot(ag, wu2_ref[...], preferred_element_type=jnp.float32, precision=lax.Precision.HIGHEST)
    lat = jnp.maximum(lat, 0.0)
    lat_ref[...] = lat
    out_ref[...] = (jnp.dot(ne, wd1_ref[...], preferred_element_type=jnp.float32, precision=lax.Precision.HIGHEST)
                    + jnp.dot(lat, wd2_ref[...], preferred_element_type=jnp.float32, precision=lax.Precision.HIGHEST))

    @pl.when(i == 0)
    def _():
        acc_ref[...] = jnp.zeros_like(acc_ref)

    bn = lat.shape[0]
    acc_ref[...] += jnp.sum(lat.reshape(bn // 8, 8, _L), axis=0)

    @pl.when(i == ng - 1)
    def _():
        t = jnp.sum(acc_ref[...] * wt_ref[...]) / n_total
        term_ref[...] = jnp.full((1, 1), t, jnp.float32)


def _decode(ne, agg, wu1, wu2, wd1, wd2, wt, bn):
    n = ne.shape[0]
    nf = wd1.shape[1]
    grid = n // bn
    return pl.pallas_call(
        functools.partial(_decode_body, float(n)),
        grid=(grid,),
        in_specs=[
            pl.BlockSpec((bn, _L), lambda i: (i, 0)),
            pl.BlockSpec((bn, _L), lambda i: (i, 0)),
            pl.BlockSpec((_L, _L), lambda i: (0, 0)),
            pl.BlockSpec((_L, _L), lambda i: (0, 0)),
            pl.BlockSpec((_L, nf), lambda i: (0, 0)),
            pl.BlockSpec((_L, nf), lambda i: (0, 0)),
            pl.BlockSpec((1, _L), lambda i: (0, 0)),
        ],
        out_specs=[
            pl.BlockSpec((bn, nf), lambda i: (i, 0)),
            pl.BlockSpec((bn, _L), lambda i: (i, 0)),
            pl.BlockSpec((1, 1), lambda i: (0, 0)),
        ],
        out_shape=[
            jax.ShapeDtypeStruct((n, nf), jnp.float32),
            jax.ShapeDtypeStruct((n, _L), jnp.float32),
            jax.ShapeDtypeStruct((1, 1), jnp.float32),
        ],
        scratch_shapes=[pltpu.VMEM((8, _L), jnp.float32)],
    )(ne, agg, wu1, wu2, wd1, wd2, wt)


def _sc_edge(ps, pd, src2, dst2, ef1, ab, zrows, half, half_pad, cpt):
    n = ps.shape[0]
    mesh = plsc.VectorSubcoreMesh(core_axis_name="c", subcore_axis_name="s")
    zchunk = half_pad // _NSUB
    wchunk = -(-(half // _NSUB) // 8) * 8
    wlast = half - (_NSUB - 1) * wchunk
    nsg = _CH // 128

    @functools.partial(
        pl.kernel,
        mesh=mesh,
        out_type=jax.ShapeDtypeStruct((n, _L), jnp.float32),
        compiler_params=pltpu.CompilerParams(needs_layout_passes=False,
                                             use_tc_tiling_on_sc=False),
        scratch_types=[
            pltpu.VMEM_SHARED((half_pad, _L), jnp.float32),
            pltpu.VMEM((nsg, 128), jnp.int32),
            pltpu.VMEM((nsg, 128), jnp.int32),
            pltpu.VMEM((_CH,), jnp.float32),
            pltpu.VMEM((_CH, _L), jnp.float32),
            pltpu.VMEM((_CH, _L), jnp.float32),
            pltpu.VMEM((nsg, 128), jnp.int32),
            pltpu.VMEM((4, 16), jnp.float32),
            pltpu.SemaphoreType.DMA,
        ],
    )
    def edge_kernel(ps_hbm, pd_hbm, src_hbm, dst_hbm, ef_hbm, ab_hbm, z_hbm,
                    out_hbm, agg, sidx, didx, efb, gs, msg, lidx, abv,
                    gsem):
        c = lax.axis_index("c")
        s = lax.axis_index("s")
        node_base = c * half

        pltpu.sync_copy(z_hbm, agg.at[pl.ds(s * zchunk, zchunk)])
        pltpu.sync_copy(ab_hbm, abv)
        plsc.subcore_barrier()

        a0 = abv[0, :]
        a1 = abv[1, :]
        b0 = abv[2, :]
        b1 = abv[3, :]

        row_base = s * (cpt * nsg)

        def chunk_body(k, carry):
            roff = row_base + k * nsg
            pltpu.sync_copy(src_hbm.at[pl.ds(roff, nsg)], sidx)
            pltpu.sync_copy(dst_hbm.at[pl.ds(roff, nsg)], didx)
            pltpu.sync_copy(ef_hbm.at[pl.ds(roff * 128, _CH)], efb)

            descs = []
            for j in range(nsg):
                descs.append(pltpu.async_copy(
                    ps_hbm.at[sidx.at[j]], gs.at[pl.ds(j * 128, 128)], gsem))
                descs.append(pltpu.async_copy(
                    pd_hbm.at[didx.at[j]], msg.at[pl.ds(j * 128, 128)], gsem))

            for j in range(nsg):
                for g in range(8):
                    d = didx[j, pl.ds(g * 16, 16)]
                    loc = d - node_base
                    oob = (loc < 0) | (loc >= half)
                    lidx[j, pl.ds(g * 16, 16)] = jnp.where(oob, half, loc)

            for dsc in descs:
                dsc.wait()

            def edge_body(e, carry2):
                efv = plsc.load_gather(efb, [jnp.full((16,), e, jnp.int32)])
                cond = efv >= 0.0
                q0 = efv * jnp.where(cond, a0, b0)
                q1 = efv * jnp.where(cond, a1, b1)
                s0 = gs[e, pl.ds(0, 16)] + msg[e, pl.ds(0, 16)]
                s1 = gs[e, pl.ds(16, 16)] + msg[e, pl.ds(16, 16)]
                msg[e, pl.ds(0, 16)] = jnp.maximum(s0 + q0, 0.0)
                msg[e, pl.ds(16, 16)] = jnp.maximum(s1 + q1, 0.0)
                return carry2

            lax.fori_loop(0, _CH, edge_body, 0)

            for j in range(nsg):
                pltpu.sync_copy(msg.at[pl.ds(j * 128, 128)],
                                agg.at[lidx.at[j]], add=True)
            return carry

        lax.fori_loop(0, cpt, chunk_body, 0)
        plsc.subcore_barrier()

        @pl.when(s < _NSUB - 1)
        def _():
            pltpu.sync_copy(agg.at[pl.ds(s * wchunk, wchunk)],
                            out_hbm.at[pl.ds(node_base + s * wchunk, wchunk)])

        @pl.when(s == _NSUB - 1)
        def _():
            base = (_NSUB - 1) * wchunk
            pltpu.sync_copy(agg.at[pl.ds(base, wlast)],
                            out_hbm.at[pl.ds(node_base + base, wlast)])

    return edge_kernel(ps, pd, src2, dst2, ef1, ab, zrows)


def kernel(node_features, edge_features, edge_index, last_latent,
           W_node, W_edge, W_msg, W_upd, W_dec, W_term):
    n = node_features.shape[0]
    e = edge_features.shape[0]
    f32 = jnp.float32

    half = n // 2
    half_pad = half + 8 * _NSUB
    cpt = -(-e // (_NSUB * _CH))
    e_pad = _NSUB * cpt * _CH

    x = jnp.concatenate([node_features, last_latent], axis=1)
    ws, wd, we = W_msg[:_L], W_msg[_L:2 * _L], W_msg[2 * _L:]
    ne, ps, pd = _encode(x, W_node, ws, wd, bn=2000)

    w = W_edge[0]
    u = jnp.where(w >= 0, w, 0.01 * w)
    v = jnp.where(w < 0, w, 0.01 * w)
    a = jnp.dot(u, we, precision=lax.Precision.HIGHEST)
    b = jnp.dot(v, we, precision=lax.Precision.HIGHEST)
    ab = jnp.concatenate([a, b]).reshape(4, 16)

    pad = e_pad - e
    src = jnp.concatenate([edge_index[0], jnp.zeros((pad,), jnp.int32)])
    dst = jnp.concatenate([edge_index[1], jnp.full((pad,), n, jnp.int32)])
    ef = jnp.concatenate([edge_features[:, 0], jnp.zeros((pad,), f32)])
    src2 = src.reshape(-1, 128)
    dst2 = dst.reshape(-1, 128)
    zrows = jnp.zeros((half_pad // _NSUB, _L), f32)

    agg = _sc_edge(ps, pd, src2, dst2, ef, ab, zrows, half, half_pad, cpt)

    out, lat, term = _decode(ne, agg, W_upd[:_L], W_upd[_L:], W_dec[:_L],
                             W_dec[_L:], W_term.reshape(1, _L), bn=2000)
    return (out, lat, term.reshape(1))
